# Initial kernel scaffold; baseline (speedup 1.0000x reference)
import jax
import jax.numpy as jnp
from jax import lax
from jax.experimental import pallas as pl
from jax.experimental.pallas import tpu as pltpu
from jax.experimental.pallas import tpu_sc as plsc

B, M, N = 4, 4096, 4096
NNZ = 167772
NNZ_PAD = 167808
SHARD = NNZ_PAD // 8
CHUNKS = SHARD // 16
W_PRIMAL, W_DUAL, W_STAT, W_COMP = 0.1, 0.1, 0.6, 0.2
INV_MB = 1.0 / float(M * B)


def _sc_body(x_hbm, lam_hbm, vals_hbm, rows_hbm, cols_hbm, b_hbm, c_hbm,
             out_hbm,
             vals_v, rows_v, cols_v, x_v, lam_v, acc_v, tmp_v, din_v, out_v,
             acc_sh):
    c = lax.axis_index("c")
    s = lax.axis_index("s")
    p_local = s // 8
    p = 2 * c + p_local
    j = s - p_local * 8
    rowid = c * 16 + s

    pltpu.sync_copy(x_hbm.at[pl.ds(p * N, N)], x_v)
    pltpu.sync_copy(lam_hbm.at[pl.ds(p * M, M)], lam_v)
    base = j * SHARD
    pltpu.sync_copy(vals_hbm.at[p, pl.ds(base, SHARD)], vals_v)
    pltpu.sync_copy(rows_hbm.at[p, pl.ds(base, SHARD)], rows_v)
    pltpu.sync_copy(cols_hbm.at[p, pl.ds(base, SHARD)], cols_v)

    zero16 = jnp.zeros((16,), jnp.float32)

    def zbody(k, carry):
        acc_v[pl.ds(k * 16, 16)] = zero16
        return carry

    lax.fori_loop(0, (M + N) // 16, zbody, 0)

    def mbody(k, carry):
        o = k * 16
        idx_r = rows_v[pl.ds(o, 16)]
        idx_c = cols_v[pl.ds(o, 16)]
        v = vals_v[pl.ds(o, 16)]
        xg = plsc.load_gather(x_v, [idx_c])
        plsc.addupdate_scatter(acc_v, [idx_r], v * xg)
        lg = plsc.load_gather(lam_v, [idx_r])
        plsc.addupdate_scatter(acc_v, [idx_c + N], v * lg)
        return carry

    lax.fori_loop(0, CHUNKS, mbody, 0)

    pltpu.sync_copy(acc_v, acc_sh.at[s])
    plsc.subcore_barrier()

    pltpu.sync_copy(
        acc_sh.at[pl.ds(p_local * 8, 8), pl.ds(j * 1024, 1024)], tmp_v)

    @pl.when(j < 4)
    def _():
        pltpu.sync_copy(b_hbm.at[p, pl.ds(j * 1024, 1024)], din_v)

        def sbody(k, carry):
            sp, sd, sm = carry
            o = k * 16
            a16 = tmp_v[0, pl.ds(o, 16)]
            for t in range(1, 8):
                a16 = a16 + tmp_v[t, pl.ds(o, 16)]
            bb = din_v[pl.ds(o, 16)]
            ll = lam_v[pl.ds(j * 1024 + o, 16)]
            axmb = a16 - bb
            rp = jnp.maximum(axmb, 0.0)
            rn = jnp.maximum(-ll, 0.0)
            cm = ll * axmb
            return (sp + rp * rp, sd + rn * rn, sm + cm * cm)

        sp, sd, sm = lax.fori_loop(0, 64, sbody, (zero16, zero16, zero16))
        val = (W_PRIMAL * jnp.sum(sp) + W_DUAL * jnp.sum(sd)
               + W_COMP * jnp.sum(sm)) * INV_MB
        out_v[...] = jnp.full((16,), 1.0, jnp.float32) * val
        pltpu.sync_copy(out_v, out_hbm.at[rowid])

    @pl.when(j >= 4)
    def _():
        pltpu.sync_copy(c_hbm.at[p, pl.ds(j * 1024 - N, 1024)], din_v)

        def sbody(k, st):
            o = k * 16
            a16 = tmp_v[0, pl.ds(o, 16)]
            for t in range(1, 8):
                a16 = a16 + tmp_v[t, pl.ds(o, 16)]
            cc = din_v[pl.ds(o, 16)]
            r = a16 + cc
            return st + r * r

        st = lax.fori_loop(0, 64, sbody, zero16)
        val = W_STAT * jnp.sum(st) * INV_MB
        out_v[...] = jnp.full((16,), 1.0, jnp.float32) * val
        pltpu.sync_copy(out_v, out_hbm.at[rowid])


@jax.jit
def _run(x_hat, lam_hat, vals_p, rows_p, cols_p, b_pad, c_pad):
    mesh = plsc.VectorSubcoreMesh(core_axis_name="c", subcore_axis_name="s")
    kfn = pl.kernel(
        _sc_body,
        out_type=jax.ShapeDtypeStruct((32, 16), jnp.float32),
        mesh=mesh,
        scratch_types=[
            pltpu.VMEM((SHARD,), jnp.float32),
            pltpu.VMEM((SHARD,), jnp.int32),
            pltpu.VMEM((SHARD,), jnp.int32),
            pltpu.VMEM((N,), jnp.float32),
            pltpu.VMEM((M,), jnp.float32),
            pltpu.VMEM((M + N,), jnp.float32),
            pltpu.VMEM((8, 1024), jnp.float32),
            pltpu.VMEM((1024,), jnp.float32),
            pltpu.VMEM((16,), jnp.float32),
            pltpu.VMEM_SHARED((16, M + N), jnp.float32),
        ],
    )
    out = kfn(x_hat, lam_hat, vals_p, rows_p, cols_p, b_pad, c_pad)
    return jnp.sum(out[:, 0])


def kernel(x_hat, lam_hat, A_vals, A_rows, A_cols, b_pad, c_pad):
    pad = NNZ_PAD - NNZ
    vals_p = jnp.pad(A_vals, ((0, 0), (0, pad)))
    rows_p = jnp.pad(A_rows, ((0, 0), (0, pad)))
    cols_p = jnp.pad(A_cols, ((0, 0), (0, pad)))
    return _run(x_hat, lam_hat, vals_p, rows_p, cols_p, b_pad, c_pad)


# speedup vs baseline: 287.1772x; 287.1772x over previous
import jax
import jax.numpy as jnp
from jax import lax
from jax.experimental import pallas as pl
from jax.experimental.pallas import tpu as pltpu
from jax.experimental.pallas import tpu_sc as plsc

B, M, N = 4, 4096, 4096
NNZ = 167772
NNZ_PAD = 167808
SHARD = NNZ_PAD // 8
CHUNKS = SHARD // 16
MN = M + N
W_PRIMAL, W_DUAL, W_STAT, W_COMP = 0.1, 0.1, 0.6, 0.2
INV_MB = 1.0 / float(M * B)


def _sc_body(x_hbm, lam_hbm, vals_hbm, rows_hbm, cols_hbm, b_hbm, c_hbm,
             out_hbm,
             vals_v, rows_v, cols_v, x_v, lam_v, acc_v, tmp_v, din_v, out_v,
             acc_sh):
    c = lax.axis_index("c")
    s = lax.axis_index("s")
    p_local = s // 8
    p = 2 * c + p_local
    j = s - p_local * 8
    rowid = c * 16 + s

    pltpu.sync_copy(x_hbm.at[pl.ds(p * N, N)], x_v)
    pltpu.sync_copy(lam_hbm.at[pl.ds(p * M, M)], lam_v)
    base = p * NNZ_PAD + j * SHARD
    pltpu.sync_copy(vals_hbm.at[pl.ds(base, SHARD)], vals_v)
    pltpu.sync_copy(rows_hbm.at[pl.ds(base, SHARD)], rows_v)
    pltpu.sync_copy(cols_hbm.at[pl.ds(base, SHARD)], cols_v)

    zero16 = jnp.zeros((16,), jnp.float32)

    def zbody(k, carry):
        acc_v[pl.ds(k * 16, 16)] = zero16
        return carry

    lax.fori_loop(0, MN // 16, zbody, 0)

    def mbody(k, carry):
        o = k * 16
        idx_r = rows_v[pl.ds(o, 16)]
        idx_c = cols_v[pl.ds(o, 16)]
        v = vals_v[pl.ds(o, 16)]
        xg = plsc.load_gather(x_v, [idx_c])
        plsc.addupdate_scatter(acc_v, [idx_r], v * xg)
        lg = plsc.load_gather(lam_v, [idx_r])
        plsc.addupdate_scatter(acc_v, [idx_c + N], v * lg)
        return carry

    lax.fori_loop(0, CHUNKS, mbody, 0)

    pltpu.sync_copy(acc_v, acc_sh.at[pl.ds(s * MN, MN)])
    plsc.subcore_barrier()

    sh_base = p_local * 8 * MN + j * 1024
    for t in range(8):
        pltpu.sync_copy(acc_sh.at[pl.ds(sh_base + t * MN, 1024)],
                        tmp_v.at[pl.ds(t * 1024, 1024)])

    @pl.when(j < 4)
    def _():
        pltpu.sync_copy(b_hbm.at[pl.ds(p * M + j * 1024, 1024)], din_v)

        def sbody(k, carry):
            sp, sd, sm = carry
            o = k * 16
            a16 = tmp_v[pl.ds(o, 16)]
            for t in range(1, 8):
                a16 = a16 + tmp_v[pl.ds(t * 1024 + o, 16)]
            bb = din_v[pl.ds(o, 16)]
            ll = lam_v[pl.ds(j * 1024 + o, 16)]
            axmb = a16 - bb
            rp = jnp.maximum(axmb, 0.0)
            rn = jnp.maximum(-ll, 0.0)
            cm = ll * axmb
            return (sp + rp * rp, sd + rn * rn, sm + cm * cm)

        sp, sd, sm = lax.fori_loop(0, 64, sbody, (zero16, zero16, zero16))
        val = (W_PRIMAL * jnp.sum(sp) + W_DUAL * jnp.sum(sd)
               + W_COMP * jnp.sum(sm)) * INV_MB
        out_v[...] = jnp.full((16,), 1.0, jnp.float32) * val
        pltpu.sync_copy(out_v, out_hbm.at[pl.ds(rowid * 16, 16)])

    @pl.when(j >= 4)
    def _():
        pltpu.sync_copy(c_hbm.at[pl.ds(p * N + j * 1024 - N, 1024)], din_v)

        def sbody(k, st):
            o = k * 16
            a16 = tmp_v[pl.ds(o, 16)]
            for t in range(1, 8):
                a16 = a16 + tmp_v[pl.ds(t * 1024 + o, 16)]
            cc = din_v[pl.ds(o, 16)]
            r = a16 + cc
            return st + r * r

        st = lax.fori_loop(0, 64, sbody, zero16)
        val = W_STAT * jnp.sum(st) * INV_MB
        out_v[...] = jnp.full((16,), 1.0, jnp.float32) * val
        pltpu.sync_copy(out_v, out_hbm.at[pl.ds(rowid * 16, 16)])


@jax.jit
def _run(x_hat, lam_hat, vals_p, rows_p, cols_p, b_flat, c_flat):
    mesh = plsc.VectorSubcoreMesh(core_axis_name="c", subcore_axis_name="s")
    kfn = pl.kernel(
        _sc_body,
        out_type=jax.ShapeDtypeStruct((32 * 16,), jnp.float32),
        mesh=mesh,
        compiler_params=pltpu.CompilerParams(needs_layout_passes=False),
        scratch_types=[
            pltpu.VMEM((SHARD,), jnp.float32),
            pltpu.VMEM((SHARD,), jnp.int32),
            pltpu.VMEM((SHARD,), jnp.int32),
            pltpu.VMEM((N,), jnp.float32),
            pltpu.VMEM((M,), jnp.float32),
            pltpu.VMEM((MN,), jnp.float32),
            pltpu.VMEM((8 * 1024,), jnp.float32),
            pltpu.VMEM((1024,), jnp.float32),
            pltpu.VMEM((16,), jnp.float32),
            pltpu.VMEM_SHARED((16 * MN,), jnp.float32),
        ],
    )
    out = kfn(x_hat, lam_hat, vals_p, rows_p, cols_p, b_flat, c_flat)
    return jnp.sum(out.reshape(32, 16)[:, 0])


def kernel(x_hat, lam_hat, A_vals, A_rows, A_cols, b_pad, c_pad):
    pad = NNZ_PAD - NNZ
    vals_p = jnp.pad(A_vals, ((0, 0), (0, pad))).reshape(-1)
    rows_p = jnp.pad(A_rows, ((0, 0), (0, pad))).reshape(-1)
    cols_p = jnp.pad(A_cols, ((0, 0), (0, pad))).reshape(-1)
    return _run(x_hat, lam_hat, vals_p, rows_p, cols_p,
                b_pad.reshape(-1), c_pad.reshape(-1))


# speedup vs baseline: 384.0914x; 1.3375x over previous
import jax
import jax.numpy as jnp
from jax import lax
from jax.experimental import pallas as pl
from jax.experimental.pallas import tpu as pltpu
from jax.experimental.pallas import tpu_sc as plsc

B, M, N = 4, 4096, 4096
NNZ = 167772
NNZ_PAD = 167808
SHARD = NNZ_PAD // 8
CHUNKS = SHARD // 16
MN = M + N
W_PRIMAL, W_DUAL, W_STAT, W_COMP = 0.1, 0.1, 0.6, 0.2
INV_MB = 1.0 / float(M * B)


def _sc_body(x_hbm, lam_hbm, vals_hbm, rows_hbm, cols_hbm, b_hbm, c_hbm,
             out_hbm,
             vals_v, rows_v, cols_v, x_v, lam_v, acc_v, tmp_v, din_v, out_v,
             acc_sh):
    c = lax.axis_index("c")
    s = lax.axis_index("s")
    p_local = s // 8
    p = 2 * c + p_local
    j = s - p_local * 8
    rowid = c * 16 + s

    pltpu.sync_copy(x_hbm.at[pl.ds(p * N, N)], x_v)
    pltpu.sync_copy(lam_hbm.at[pl.ds(p * M, M)], lam_v)
    base = p * NNZ_PAD + j * SHARD
    pltpu.sync_copy(vals_hbm.at[pl.ds(base, SHARD)], vals_v)
    pltpu.sync_copy(rows_hbm.at[pl.ds(base, SHARD)], rows_v)
    pltpu.sync_copy(cols_hbm.at[pl.ds(base, SHARD)], cols_v)

    zero16 = jnp.zeros((16,), jnp.float32)

    @plsc.parallel_loop(0, MN // 16, unroll=8)
    def _(k):
        acc_v[pl.ds(k * 16, 16)] = zero16

    @plsc.parallel_loop(0, CHUNKS, unroll=8)
    def _(k):
        o = k * 16
        idx_r = rows_v[pl.ds(o, 16)]
        idx_c = cols_v[pl.ds(o, 16)]
        v = vals_v[pl.ds(o, 16)]
        xg = plsc.load_gather(x_v, [idx_c])
        plsc.addupdate_scatter(acc_v, [idx_r], v * xg)
        lg = plsc.load_gather(lam_v, [idx_r])
        plsc.addupdate_scatter(acc_v, [idx_c + N], v * lg)

    pltpu.sync_copy(acc_v, acc_sh.at[pl.ds(s * MN, MN)])
    plsc.subcore_barrier()

    sh_base = p_local * 8 * MN + j * 1024
    for t in range(8):
        pltpu.sync_copy(acc_sh.at[pl.ds(sh_base + t * MN, 1024)],
                        tmp_v.at[pl.ds(t * 1024, 1024)])

    @pl.when(j < 4)
    def _():
        pltpu.sync_copy(b_hbm.at[pl.ds(p * M + j * 1024, 1024)], din_v)

        def sbody(k, carry):
            sp, sd, sm = carry
            o = k * 16
            a16 = tmp_v[pl.ds(o, 16)]
            for t in range(1, 8):
                a16 = a16 + tmp_v[pl.ds(t * 1024 + o, 16)]
            bb = din_v[pl.ds(o, 16)]
            ll = lam_v[pl.ds(j * 1024 + o, 16)]
            axmb = a16 - bb
            rp = jnp.maximum(axmb, 0.0)
            rn = jnp.maximum(-ll, 0.0)
            cm = ll * axmb
            return (sp + rp * rp, sd + rn * rn, sm + cm * cm)

        sp, sd, sm = lax.fori_loop(0, 64, sbody, (zero16, zero16, zero16))
        val = (W_PRIMAL * jnp.sum(sp) + W_DUAL * jnp.sum(sd)
               + W_COMP * jnp.sum(sm)) * INV_MB
        out_v[...] = jnp.full((16,), 1.0, jnp.float32) * val
        pltpu.sync_copy(out_v, out_hbm.at[pl.ds(rowid * 16, 16)])

    @pl.when(j >= 4)
    def _():
        pltpu.sync_copy(c_hbm.at[pl.ds(p * N + j * 1024 - N, 1024)], din_v)

        def sbody(k, st):
            o = k * 16
            a16 = tmp_v[pl.ds(o, 16)]
            for t in range(1, 8):
                a16 = a16 + tmp_v[pl.ds(t * 1024 + o, 16)]
            cc = din_v[pl.ds(o, 16)]
            r = a16 + cc
            return st + r * r

        st = lax.fori_loop(0, 64, sbody, zero16)
        val = W_STAT * jnp.sum(st) * INV_MB
        out_v[...] = jnp.full((16,), 1.0, jnp.float32) * val
        pltpu.sync_copy(out_v, out_hbm.at[pl.ds(rowid * 16, 16)])


@jax.jit
def _run(x_hat, lam_hat, vals_p, rows_p, cols_p, b_flat, c_flat):
    mesh = plsc.VectorSubcoreMesh(core_axis_name="c", subcore_axis_name="s")
    kfn = pl.kernel(
        _sc_body,
        out_type=jax.ShapeDtypeStruct((32 * 16,), jnp.float32),
        mesh=mesh,
        compiler_params=pltpu.CompilerParams(needs_layout_passes=False),
        scratch_types=[
            pltpu.VMEM((SHARD,), jnp.float32),
            pltpu.VMEM((SHARD,), jnp.int32),
            pltpu.VMEM((SHARD,), jnp.int32),
            pltpu.VMEM((N,), jnp.float32),
            pltpu.VMEM((M,), jnp.float32),
            pltpu.VMEM((MN,), jnp.float32),
            pltpu.VMEM((8 * 1024,), jnp.float32),
            pltpu.VMEM((1024,), jnp.float32),
            pltpu.VMEM((16,), jnp.float32),
            pltpu.VMEM_SHARED((16 * MN,), jnp.float32),
        ],
    )
    out = kfn(x_hat, lam_hat, vals_p, rows_p, cols_p, b_flat, c_flat)
    return jnp.sum(out.reshape(32, 16)[:, 0])


def kernel(x_hat, lam_hat, A_vals, A_rows, A_cols, b_pad, c_pad):
    pad = NNZ_PAD - NNZ
    vals_p = jnp.pad(A_vals, ((0, 0), (0, pad))).reshape(-1)
    rows_p = jnp.pad(A_rows, ((0, 0), (0, pad))).reshape(-1)
    cols_p = jnp.pad(A_cols, ((0, 0), (0, pad))).reshape(-1)
    return _run(x_hat, lam_hat, vals_p, rows_p, cols_p,
                b_pad.reshape(-1), c_pad.reshape(-1))
